# baseline (speedup 1.0000x reference)
import functools

import jax
import jax.numpy as jnp
from jax import lax
from jax.experimental import pallas as pl
from jax.experimental.pallas import tpu as pltpu
from jax.experimental.pallas import tpu_sc as plsc

B, L, NF = 16384, 50, 5
VOCAB, DIM = 100000, 32
N = B * L

NC, NS, LANES = 2, 16, 16
NW = NC * NS
N_PER_W = N // NW

C = 256
NI = NF * C
G_IDX = 128
N_GROUPS = NI // G_IDX
N_CHUNKS = N_PER_W // C


def _body(gx_hbm, tab_hbm, out_hbm,
          xv_a, xv_b, rows_a, rows_b, outv_a, outv_b,
          sem_xa, sem_xb, sem_ga, sem_gb, sem_oa, sem_ob):
    wid = lax.axis_index("s") * NC + lax.axis_index("c")
    base = wid * N_PER_W

    def xload(chunk, xv, sem):
        pltpu.async_copy(gx_hbm.at[pl.ds((base + chunk * C) * NF, NI)], xv, sem)

    def xwait(xv, sem):
        pltpu.make_async_copy(gx_hbm.at[pl.ds(0, NI)], xv, sem).wait()

    def fire(xv, rows, sem):
        def gather_body(g, carry):
            pltpu.async_copy(
                tab_hbm.at[xv.at[pl.ds(g * G_IDX, G_IDX)]],
                rows.at[pl.ds(g * G_IDX, G_IDX), :],
                sem,
            )
            return carry

        lax.fori_loop(0, N_GROUPS, gather_body, None)

    def gwait(rows, sem):
        pltpu.make_async_copy(tab_hbm.at[pl.ds(0, NI)], rows, sem).wait()

    def reduce(rows, outv):
        @plsc.parallel_loop(0, C, unroll=4)
        def red_body(c):
            r0 = c * NF
            lo = rows[r0, pl.ds(0, LANES)]
            hi = rows[r0, pl.ds(LANES, LANES)]
            for t in range(1, NF):
                lo = lo + rows[r0 + t, pl.ds(0, LANES)]
                hi = hi + rows[r0 + t, pl.ds(LANES, LANES)]
            outv[c, pl.ds(0, LANES)] = lo
            outv[c, pl.ds(LANES, LANES)] = hi

    def owrite(chunk, outv, sem):
        pltpu.async_copy(outv, out_hbm.at[pl.ds(base + chunk * C, C), :], sem)

    def owait(outv, sem):
        pltpu.make_async_copy(outv, out_hbm.at[pl.ds(base, C), :], sem).wait()

    xload(0, xv_a, sem_xa)
    xwait(xv_a, sem_xa)
    fire(xv_a, rows_a, sem_ga)
    xload(1, xv_b, sem_xb)

    def loop(kk, _):
        c0 = 2 * kk
        xwait(xv_b, sem_xb)
        fire(xv_b, rows_b, sem_gb)

        gwait(rows_a, sem_ga)

        @pl.when(kk > 0)
        def _():
            owait(outv_a, sem_oa)

        reduce(rows_a, outv_a)
        owrite(c0, outv_a, sem_oa)

        @pl.when(c0 + 2 < N_CHUNKS)
        def _():
            xload(c0 + 2, xv_a, sem_xa)
            xwait(xv_a, sem_xa)
            fire(xv_a, rows_a, sem_ga)
            xload(c0 + 3, xv_b, sem_xb)

        gwait(rows_b, sem_gb)

        @pl.when(kk > 0)
        def _():
            owait(outv_b, sem_ob)

        reduce(rows_b, outv_b)
        owrite(c0 + 1, outv_b, sem_ob)
        return _

    lax.fori_loop(0, N_CHUNKS // 2, loop, None)
    owait(outv_a, sem_oa)
    owait(outv_b, sem_ob)


@jax.jit
def _run(gx_flat, tab_flat):
    mesh = plsc.VectorSubcoreMesh(core_axis_name="c", subcore_axis_name="s")
    return pl.kernel(
        _body,
        mesh=mesh,
        compiler_params=pltpu.CompilerParams(use_tc_tiling_on_sc=False),
        out_type=jax.ShapeDtypeStruct((N, DIM), jnp.float32),
        scratch_types=[
            pltpu.VMEM((NI,), jnp.int32),
            pltpu.VMEM((NI,), jnp.int32),
            pltpu.VMEM((NI, DIM), jnp.float32),
            pltpu.VMEM((NI, DIM), jnp.float32),
            pltpu.VMEM((C, DIM), jnp.float32),
            pltpu.VMEM((C, DIM), jnp.float32),
            pltpu.SemaphoreType.DMA,
            pltpu.SemaphoreType.DMA,
            pltpu.SemaphoreType.DMA,
            pltpu.SemaphoreType.DMA,
            pltpu.SemaphoreType.DMA,
            pltpu.SemaphoreType.DMA,
        ],
    )(gx_flat, tab_flat)


def kernel(x, tables):
    gx_flat = (x + jnp.arange(NF, dtype=jnp.int32) * VOCAB).reshape(-1)
    tab_flat = tables.reshape(NF * VOCAB, DIM)
    out = _run(gx_flat, tab_flat)
    return out.reshape(B, L, DIM)


# speedup vs baseline: 2.3308x; 2.3308x over previous
import functools

import jax
import jax.numpy as jnp
from jax import lax
from jax.experimental import pallas as pl
from jax.experimental.pallas import tpu as pltpu
from jax.experimental.pallas import tpu_sc as plsc

B, L, NF = 16384, 50, 5
VOCAB, DIM = 100000, 32

NC, NS, LANES = 2, 16, 16
NW = NC * NS
B_PER_W = B // NW

C = 256
G_IDX = 128
N_GROUPS = C // G_IDX
N_CHUNKS = 2 * L


def _body(xt_hbm, tab_hbm, out_hbm,
          xv_a, xv_b, rows_a, rows_b, outv_a, outv_b,
          sem_xa, sem_xb, sem_ga, sem_gb, sem_oa, sem_ob):
    wid = lax.axis_index("s") * NC + lax.axis_index("c")
    b_base = wid * B_PER_W

    def chunk_lb(chunk):
        return chunk >> 1, b_base + (chunk & 1) * C

    def xload(chunk, xv, sem):
        l, b0 = chunk_lb(chunk)
        for f in range(NF):
            pltpu.async_copy(xt_hbm.at[f, l, pl.ds(b0, C)], xv.at[f], sem)

    def xwait(xv, sem):
        pltpu.make_async_copy(
            xt_hbm.at[pl.ds(0, NF), 0, pl.ds(0, C)], xv, sem).wait()

    def fire(xv, rows, sem):
        for f in range(NF):
            for g in range(N_GROUPS):
                pltpu.async_copy(
                    tab_hbm.at[f].at[xv.at[f, pl.ds(g * G_IDX, G_IDX)]],
                    rows.at[f, pl.ds(g * G_IDX, G_IDX), :],
                    sem,
                )

    def gwait(rows, sem):
        pltpu.make_async_copy(
            tab_hbm.at[pl.ds(0, NF), pl.ds(0, C), :], rows, sem).wait()

    def reduce(rows, outv):
        @plsc.parallel_loop(0, C, unroll=4)
        def red_body(c):
            lo = rows[0, c, pl.ds(0, LANES)]
            hi = rows[0, c, pl.ds(LANES, LANES)]
            for t in range(1, NF):
                lo = lo + rows[t, c, pl.ds(0, LANES)]
                hi = hi + rows[t, c, pl.ds(LANES, LANES)]
            outv[c, pl.ds(0, LANES)] = lo
            outv[c, pl.ds(LANES, LANES)] = hi

    def owrite(chunk, outv, sem):
        l, b0 = chunk_lb(chunk)
        pltpu.async_copy(outv, out_hbm.at[l, pl.ds(b0, C), :], sem)

    def owait(outv, sem):
        pltpu.make_async_copy(outv, out_hbm.at[0, pl.ds(0, C), :], sem).wait()

    xload(0, xv_a, sem_xa)
    xwait(xv_a, sem_xa)
    fire(xv_a, rows_a, sem_ga)
    xload(1, xv_b, sem_xb)

    def loop(kk, _):
        c0 = 2 * kk
        xwait(xv_b, sem_xb)
        fire(xv_b, rows_b, sem_gb)

        gwait(rows_a, sem_ga)

        @pl.when(kk > 0)
        def _():
            owait(outv_a, sem_oa)

        reduce(rows_a, outv_a)
        owrite(c0, outv_a, sem_oa)

        @pl.when(c0 + 2 < N_CHUNKS)
        def _():
            xload(c0 + 2, xv_a, sem_xa)
            xwait(xv_a, sem_xa)
            fire(xv_a, rows_a, sem_ga)
            xload(c0 + 3, xv_b, sem_xb)

        gwait(rows_b, sem_gb)

        @pl.when(kk > 0)
        def _():
            owait(outv_b, sem_ob)

        reduce(rows_b, outv_b)
        owrite(c0 + 1, outv_b, sem_ob)
        return _

    lax.fori_loop(0, N_CHUNKS // 2, loop, None)
    owait(outv_a, sem_oa)
    owait(outv_b, sem_ob)


@jax.jit
def _run(xt, tables):
    mesh = plsc.VectorSubcoreMesh(core_axis_name="c", subcore_axis_name="s")
    return pl.kernel(
        _body,
        mesh=mesh,
        compiler_params=pltpu.CompilerParams(use_tc_tiling_on_sc=False),
        out_type=jax.ShapeDtypeStruct((L, B, DIM), jnp.float32),
        scratch_types=[
            pltpu.VMEM((NF, C), jnp.int32),
            pltpu.VMEM((NF, C), jnp.int32),
            pltpu.VMEM((NF, C, DIM), jnp.float32),
            pltpu.VMEM((NF, C, DIM), jnp.float32),
            pltpu.VMEM((C, DIM), jnp.float32),
            pltpu.VMEM((C, DIM), jnp.float32),
            pltpu.SemaphoreType.DMA,
            pltpu.SemaphoreType.DMA,
            pltpu.SemaphoreType.DMA,
            pltpu.SemaphoreType.DMA,
            pltpu.SemaphoreType.DMA,
            pltpu.SemaphoreType.DMA,
        ],
    )(xt, tables)


def kernel(x, tables):
    xt = jnp.transpose(x, (2, 1, 0))
    out_lbd = _run(xt, tables)
    return jnp.transpose(out_lbd, (1, 0, 2))
